# Initial kernel scaffold; baseline (speedup 1.0000x reference)
import jax
import jax.numpy as jnp
from jax.experimental import pallas as pl


def kernel(user_embed, item_embed, rows, cols, vals):
    raise NotImplementedError("write your pallas kernel here")


# speedup vs baseline: 3.5414x; 3.5414x over previous
import jax
import jax.numpy as jnp
from jax import lax
from jax.experimental import pallas as pl
from jax.experimental.pallas import tpu as pltpu
from jax.experimental.pallas import tpu_sc as plsc

N_USERS = 5000
N_ITEMS = 5000
N = N_USERS + N_ITEMS
E = 160000
D = 256
HOPS = 3

NC = 2
NS = 16
HD = D // NC
B = 128
NB = 79
EPT = NB * B
NP = 10240
RPT = NP // NS
ZR = 128


def _tec_body(xin, cols_h, rows_h, vals_h, out, xcur,
              cols_v, rows_v, vals_v, gbuf, acc, sem):
    c = lax.axis_index("c")
    s = lax.axis_index("s")
    row0 = s * RPT
    cbase = c * NP

    pltpu.sync_copy(cols_h.at[s], cols_v)
    pltpu.sync_copy(rows_h.at[s], rows_v)
    pltpu.sync_copy(vals_h.at[s], vals_v)

    def _rebase(j, carry):
        for k in range(8):
            sl = (j, pl.ds(k * 16, 16))
            cols_v[sl] = cols_v[sl] + cbase
        return carry

    lax.fori_loop(0, NB, _rebase, 0)

    def _zfill(j, carry):
        for k in range(8):
            gbuf[j, pl.ds(k * 16, 16)] = jnp.zeros((16,), jnp.float32)
        return carry

    lax.fori_loop(0, ZR, _zfill, 0)
    for q in range(RPT // ZR):
        pltpu.sync_copy(gbuf, acc.at[pl.ds(row0 + q * ZR, ZR)])

    for q in range(RPT // ZR):
        r = row0 + q * ZR
        pltpu.sync_copy(xin.at[pl.ds(cbase + r, ZR)], gbuf)
        pltpu.sync_copy(gbuf, xcur.at[pl.ds(cbase + r, ZR)])
        pltpu.sync_copy(gbuf, out.at[pl.ds(r, ZR), 0, pl.ds(c * HD, HD)])
    plsc.subcore_barrier()

    for h in range(HOPS):
        def _batch(j, carry):
            pltpu.async_copy(xcur.at[cols_v.at[j]], gbuf, sem).wait()

            def _group(g, gcarry):
                vgrp = vals_v[pl.ds(j * B + g * 16, 16)]
                for e16 in range(16):
                    lane = jnp.full((16, 1), e16, dtype=jnp.int32)
                    vv = lax.gather(
                        vgrp, lane,
                        dimension_numbers=lax.GatherDimensionNumbers(
                            offset_dims=(), collapsed_slice_dims=(0,),
                            start_index_map=(0,)),
                        slice_sizes=(1,),
                        mode=lax.GatherScatterMode.PROMISE_IN_BOUNDS)
                    e = g * 16 + e16
                    for k in range(8):
                        sl = (e, pl.ds(k * 16, 16))
                        gbuf[sl] = gbuf[sl] * vv
                return gcarry

            lax.fori_loop(0, B // 16, _group, 0)

            pltpu.sync_copy(gbuf, acc.at[rows_v.at[j]], add=True)
            return carry

        lax.fori_loop(0, NB, _batch, 0)
        plsc.subcore_barrier()

        pltpu.sync_copy(acc.at[pl.ds(row0, RPT)],
                        xcur.at[pl.ds(cbase + row0, RPT)])
        pltpu.sync_copy(acc.at[pl.ds(row0, RPT)],
                        out.at[pl.ds(row0, RPT), h + 1, pl.ds(c * HD, HD)])
        if h + 1 < HOPS:
            lax.fori_loop(0, ZR, _zfill, 0)
            for q in range(RPT // ZR):
                pltpu.sync_copy(gbuf, acc.at[pl.ds(row0 + q * ZR, ZR)])
        plsc.subcore_barrier()


def _lgn(xin, cols3, rows3, vals2):
    mesh = plsc.VectorSubcoreMesh(core_axis_name="c", subcore_axis_name="s")
    out, _ = pl.kernel(
        _tec_body,
        out_type=(
            jax.ShapeDtypeStruct((NP, HOPS + 1, D), jnp.float32),
            jax.ShapeDtypeStruct((NC * NP, HD), jnp.float32),
        ),
        mesh=mesh,
        scratch_types=(
            pltpu.VMEM((NB, B), jnp.int32),
            pltpu.VMEM((NB, B), jnp.int32),
            pltpu.VMEM((EPT,), jnp.float32),
            pltpu.VMEM((B, HD), jnp.float32),
            pltpu.VMEM_SHARED((NP, HD), jnp.float32),
            pltpu.SemaphoreType.DMA,
        ),
    )(xin, cols3, rows3, vals2)
    return out


def kernel(user_embed, item_embed, rows, cols, vals):
    all_embed = jnp.concatenate([user_embed, item_embed], axis=0)
    ap = jnp.pad(all_embed, ((0, NP - N), (0, 0)))
    xin = jnp.concatenate([ap[:, :HD], ap[:, HD:]], axis=0)
    pad = NS * EPT - E
    rows_p = jnp.pad(rows.astype(jnp.int32), (0, pad))
    cols_p = jnp.pad(cols.astype(jnp.int32), (0, pad))
    vals_p = jnp.pad(vals, (0, pad))
    out = _lgn(
        xin,
        cols_p.reshape(NS, NB, B),
        rows_p.reshape(NS, NB, B),
        vals_p.reshape(NS, EPT),
    )
    return out[:N_USERS], out[N_USERS:N]
